# baseline (speedup 1.0000x reference)
import functools

import jax
import jax.numpy as jnp
from jax import lax
from jax.experimental import pallas as pl
from jax.experimental.pallas import tpu as pltpu
from jax.experimental.pallas import tpu_sc as plsc

N_SPINS = 20
BATCH = 16384
NC = 2
NS = 16
NW = NC * NS
B_PER_W = BATCH // NW
LANES = 16
GROUPS = B_PER_W // LANES
CHUNK = 128
NCHUNK = B_PER_W // CHUNK

_LN2 = 0.6931471805599453
_SQRT2 = 1.4142135623730951


def _log16(v):
    bits = lax.bitcast_convert_type(v, jnp.int32)
    e = (bits >> 23) - 127
    m = lax.bitcast_convert_type(
        (bits & 0x007FFFFF) | 0x3F800000, jnp.float32)
    big = m > _SQRT2
    m = jnp.where(big, m * 0.5, m)
    e = jnp.where(big, e + 1, e)
    t = (m - 1.0) / (m + 1.0)
    t2 = t * t
    poly = 2.0 * t * (1.0 + t2 * (1.0 / 3.0 + t2 * (
        1.0 / 5.0 + t2 * (1.0 / 7.0 + t2 * (1.0 / 9.0)))))
    return e.astype(jnp.float32) * _LN2 + poly


@functools.partial(
    pl.kernel,
    out_type=jax.ShapeDtypeStruct((BATCH,), jnp.float32),
    mesh=plsc.VectorSubcoreMesh(
        core_axis_name="c", subcore_axis_name="s",
        num_cores=NC, num_subcores=NS),
    scratch_types=[
        pltpu.VMEM((B_PER_W * N_SPINS,), jnp.int32),
        pltpu.VMEM((B_PER_W,), jnp.int32),
        pltpu.VMEM((B_PER_W,), jnp.float32),
        pltpu.VMEM((B_PER_W,), jnp.float32),
        pltpu.SemaphoreType.DMA,
    ],
    compiler_params=pltpu.CompilerParams(needs_layout_passes=False),
)
def _ewf_sc(x_hbm, table_hbm, out_hbm, x_v, idx_v, val_v, out_v, sem):
    wid = lax.axis_index("s") * NC + lax.axis_index("c")
    base = wid * B_PER_W

    pltpu.sync_copy(x_hbm.at[pl.ds(base * N_SPINS, B_PER_W * N_SPINS)], x_v)

    def pack_group(g, carry):
        rowoff = (g * LANES + lax.iota(jnp.int32, 16)) * N_SPINS
        acc = jnp.zeros((16,), jnp.int32)
        for j in range(N_SPINS):
            acc = acc + acc + plsc.load_gather(x_v, [rowoff + j])
        idx_v[pl.ds(g * LANES, LANES)] = acc
        return carry

    lax.fori_loop(0, GROUPS, pack_group, 0)

    handles = [
        pltpu.async_copy(
            table_hbm.at[idx_v.at[pl.ds(k * CHUNK, CHUNK)]],
            val_v.at[pl.ds(k * CHUNK, CHUNK)], sem)
        for k in range(NCHUNK)
    ]
    for h in handles:
        h.wait()

    def log_group(g, carry):
        out_v[pl.ds(g * LANES, LANES)] = _log16(val_v[pl.ds(g * LANES, LANES)])
        return carry

    lax.fori_loop(0, GROUPS, log_group, 0)

    pltpu.sync_copy(out_v, out_hbm.at[pl.ds(base, B_PER_W)])


def kernel(x, table, j1):
    del j1
    return _ewf_sc(x.reshape(-1), table)


# speedup vs baseline: 1.1297x; 1.1297x over previous
import functools

import jax
import jax.numpy as jnp
from jax import lax
from jax.experimental import pallas as pl
from jax.experimental.pallas import tpu as pltpu
from jax.experimental.pallas import tpu_sc as plsc

N_SPINS = 20
BATCH = 16384
NC = 2
NS = 16
NW = NC * NS
B_PER_W = BATCH // NW
LANES = 16
GROUPS = B_PER_W // LANES
CHUNK = 128
NCHUNK = B_PER_W // CHUNK

_LN2 = 0.6931471805599453
_SQRT2 = 1.4142135623730951


def _log16(v):
    bits = lax.bitcast_convert_type(v, jnp.int32)
    e = (bits >> 23) - 127
    m = lax.bitcast_convert_type(
        (bits & 0x007FFFFF) | 0x3F800000, jnp.float32)
    big = m > _SQRT2
    m = jnp.where(big, m * 0.5, m)
    e = jnp.where(big, e + 1, e)
    t = (m - 1.0) / (m + 1.0)
    t2 = t * t
    poly = 2.0 * t * (1.0 + t2 * (1.0 / 3.0 + t2 * (
        1.0 / 5.0 + t2 * (1.0 / 7.0 + t2 * (1.0 / 9.0)))))
    return e.astype(jnp.float32) * _LN2 + poly


@functools.partial(
    pl.kernel,
    out_type=jax.ShapeDtypeStruct((BATCH,), jnp.float32),
    mesh=plsc.VectorSubcoreMesh(
        core_axis_name="c", subcore_axis_name="s",
        num_cores=NC, num_subcores=NS),
    scratch_types=[
        pltpu.VMEM((B_PER_W, N_SPINS), jnp.int32),
        pltpu.VMEM((B_PER_W,), jnp.int32),
        pltpu.VMEM((B_PER_W,), jnp.float32),
        pltpu.VMEM((B_PER_W,), jnp.float32),
        pltpu.SemaphoreType.DMA,
    ],
    compiler_params=pltpu.CompilerParams(needs_layout_passes=False),
)
def _ewf_sc(x_hbm, table_hbm, out_hbm, x_v, idx_v, val_v, out_v, sem):
    wid = lax.axis_index("s") * NC + lax.axis_index("c")
    base = wid * B_PER_W

    pltpu.sync_copy(x_hbm.at[pl.ds(base, B_PER_W), :], x_v)

    def pack_group(g, carry):
        rows = g * LANES + lax.iota(jnp.int32, 16)
        acc = jnp.zeros((16,), jnp.int32)
        for j in range(N_SPINS):
            col = jnp.full((16,), j, jnp.int32)
            acc = acc + acc + plsc.load_gather(x_v, [rows, col])
        idx_v[pl.ds(g * LANES, LANES)] = acc
        return carry

    lax.fori_loop(0, GROUPS, pack_group, 0)

    handles = [
        pltpu.async_copy(
            table_hbm.at[idx_v.at[pl.ds(k * CHUNK, CHUNK)]],
            val_v.at[pl.ds(k * CHUNK, CHUNK)], sem)
        for k in range(NCHUNK)
    ]
    for h in handles:
        h.wait()

    def log_group(g, carry):
        out_v[pl.ds(g * LANES, LANES)] = _log16(val_v[pl.ds(g * LANES, LANES)])
        return carry

    lax.fori_loop(0, GROUPS, log_group, 0)

    pltpu.sync_copy(out_v, out_hbm.at[pl.ds(base, B_PER_W)])


def kernel(x, table, j1):
    del j1
    return _ewf_sc(x, table)
